# Initial kernel scaffold; baseline (speedup 1.0000x reference)
import jax
import jax.numpy as jnp
from jax.experimental import pallas as pl


def kernel(x, edge_index, W0, b0, W1, b1):
    raise NotImplementedError("write your pallas kernel here")


# speedup vs baseline: 37.1726x; 37.1726x over previous
import functools

import jax
import jax.numpy as jnp
from jax import lax
from jax.experimental import pallas as pl
from jax.experimental.pallas import tpu as pltpu
from jax.experimental.pallas import tpu_sc as plsc

N = 10000
C = 128
E = 320000

NC = 2
NS = 16
NW = NC * NS

NP = 10240
EP = 327680
EPT = EP // NW
CHUNK = 128
NCH = EPT // CHUNK
HNCH = NCH // 2
ROWS_PT = NP // NS

def _sc_degree_body(dst_hbm, out_hbm, acc, dbuf, ones, zeros):
    c = lax.axis_index("c")
    s = lax.axis_index("s")
    for i in range(CHUNK // 16):
        ones[pl.ds(16 * i, 16)] = jnp.ones((16,), jnp.float32)
    for i in range(ROWS_PT // 16):
        zeros[pl.ds(16 * i, 16)] = jnp.zeros((16,), jnp.float32)
    pltpu.sync_copy(zeros, acc.at[pl.ds(s * ROWS_PT, ROWS_PT)])
    plsc.subcore_barrier()
    base = (c * NS + s) * NCH
    pltpu.sync_copy(dst_hbm.at[pl.ds(base, NCH)], dbuf)

    @pl.loop(0, NCH)
    def _(j):
        pltpu.sync_copy(ones, acc.at[dbuf.at[j]], add=True)

    plsc.subcore_barrier()
    pltpu.sync_copy(acc.at[pl.ds(s * ROWS_PT, ROWS_PT)],
                    out_hbm.at[c, 0, pl.ds(s * ROWS_PT, ROWS_PT)])


def _sc_aggregate_body(xs_hbm, src_hbm, dst_hbm, out_hbm,
                       acc, sbuf, dbuf, rows0, rows1, sem0, sem1):
    c = lax.axis_index("c")
    s = lax.axis_index("s")
    pltpu.sync_copy(xs_hbm.at[pl.ds(s * ROWS_PT, ROWS_PT)],
                    acc.at[pl.ds(s * ROWS_PT, ROWS_PT)])
    plsc.subcore_barrier()
    base = (c * NS + s) * NCH

    for h in range(2):
        hbase = base + h * HNCH
        pltpu.sync_copy(src_hbm.at[pl.ds(hbase, HNCH)], sbuf)
        pltpu.sync_copy(dst_hbm.at[pl.ds(hbase, HNCH)], dbuf)

        pltpu.async_copy(xs_hbm.at[sbuf.at[0]], rows0, sem0)

        @pl.loop(0, HNCH, step=2)
        def _(j):
            pltpu.async_copy(xs_hbm.at[sbuf.at[j + 1]], rows1, sem1)
            pltpu.make_async_copy(xs_hbm.at[sbuf.at[j]], rows0, sem0).wait()
            pltpu.sync_copy(rows0, acc.at[dbuf.at[j]], add=True)

            @pl.when(j + 2 < HNCH)
            def _():
                pltpu.async_copy(xs_hbm.at[sbuf.at[j + 2]], rows0, sem0)

            pltpu.make_async_copy(xs_hbm.at[sbuf.at[j + 1]], rows1, sem1).wait()
            pltpu.sync_copy(rows1, acc.at[dbuf.at[j + 1]], add=True)

    plsc.subcore_barrier()
    pltpu.sync_copy(acc.at[pl.ds(s * ROWS_PT, ROWS_PT)],
                    out_hbm.at[c, pl.ds(s * ROWS_PT, ROWS_PT)])


@functools.lru_cache(maxsize=None)
def _sc_kernels():
    mesh = plsc.VectorSubcoreMesh(
        core_axis_name="c", subcore_axis_name="s",
        num_cores=NC, num_subcores=NS)
    sc_degree = pl.kernel(
        _sc_degree_body,
        out_type=jax.ShapeDtypeStruct((NC, 1, NP), jnp.float32),
        mesh=mesh,
        scratch_types=[
            pltpu.VMEM_SHARED((NP,), jnp.float32),
            pltpu.VMEM((NCH, CHUNK), jnp.int32),
            pltpu.VMEM((CHUNK,), jnp.float32),
            pltpu.VMEM((ROWS_PT,), jnp.float32),
        ],
    )
    sc_aggregate = pl.kernel(
        _sc_aggregate_body,
        out_type=jax.ShapeDtypeStruct((NC, NP, C), jnp.float32),
        mesh=mesh,
        scratch_types=[
            pltpu.VMEM_SHARED((NP, C), jnp.float32),
            pltpu.VMEM((HNCH, CHUNK), jnp.int32),
            pltpu.VMEM((HNCH, CHUNK), jnp.int32),
            pltpu.VMEM((CHUNK, C), jnp.float32),
            pltpu.VMEM((CHUNK, C), jnp.float32),
            pltpu.SemaphoreType.DMA,
            pltpu.SemaphoreType.DMA,
        ],
    )
    return sc_degree, sc_aggregate


def _tc_scale_body(deg_ref, x_ref, xs_ref, dis_ref):
    deg = deg_ref[:, 0:1] + deg_ref[:, 1:2] + 1.0
    dis = lax.rsqrt(deg)
    dis_ref[...] = dis
    xs_ref[...] = x_ref[...] * dis


def _tc_scale(deg_parts, xp):
    return pl.pallas_call(
        _tc_scale_body,
        out_shape=(
            jax.ShapeDtypeStruct((NP, C), jnp.float32),
            jax.ShapeDtypeStruct((NP, 1), jnp.float32),
        ),
    )(deg_parts, xp)


def _tc_layer_body(parts_ref, xs_ref, dis_ref, res_ref, w_ref, b_ref,
                   out_ref, xsn_ref):
    dis = dis_ref[...]
    u = (parts_ref[0] + parts_ref[1] - xs_ref[...]) * dis
    y = jnp.dot(u, w_ref[...], preferred_element_type=jnp.float32) + b_ref[...]
    a = y[:, :C]
    g = y[:, C:]
    o = a * jax.nn.sigmoid(g) + res_ref[...]
    out_ref[...] = o
    xsn_ref[...] = o * dis


def _tc_layer(parts, xs, dis, res, w, b2d):
    r = 1280
    grid = NP // r
    return pl.pallas_call(
        _tc_layer_body,
        grid=(grid,),
        in_specs=[
            pl.BlockSpec((NC, r, C), lambda i: (0, i, 0)),
            pl.BlockSpec((r, C), lambda i: (i, 0)),
            pl.BlockSpec((r, 1), lambda i: (i, 0)),
            pl.BlockSpec((r, C), lambda i: (i, 0)),
            pl.BlockSpec((C, 2 * C), lambda i: (0, 0)),
            pl.BlockSpec((1, 2 * C), lambda i: (0, 0)),
        ],
        out_specs=(
            pl.BlockSpec((r, C), lambda i: (i, 0)),
            pl.BlockSpec((r, C), lambda i: (i, 0)),
        ),
        out_shape=(
            jax.ShapeDtypeStruct((NP, C), jnp.float32),
            jax.ShapeDtypeStruct((NP, C), jnp.float32),
        ),
    )(parts, xs, dis, res, w, b2d)


def kernel(x, edge_index, W0, b0, W1, b1):
    src = edge_index[0]
    dst = edge_index[1]
    npad = EP - E
    pad_idx = N + (jnp.arange(npad, dtype=jnp.int32) % (NP - N))
    srcp = jnp.concatenate([src, pad_idx]).reshape(EP // CHUNK, CHUNK)
    dstp = jnp.concatenate([dst, pad_idx]).reshape(EP // CHUNK, CHUNK)
    xp = jnp.zeros((NP, C), jnp.float32).at[:N].set(x)

    sc_degree, sc_aggregate = _sc_kernels()
    deg_parts = sc_degree(dstp)
    deg_parts = jnp.transpose(deg_parts[:, 0, :])
    xs1, dis = _tc_scale(deg_parts, xp)
    parts1 = sc_aggregate(xs1, srcp, dstp)
    x1, xs2 = _tc_layer(parts1, xs1, dis, xp,
                        W0, b0.reshape(1, 2 * C))
    parts2 = sc_aggregate(xs2, srcp, dstp)
    x2, _ = _tc_layer(parts2, xs2, dis, x1,
                      W1, b1.reshape(1, 2 * C))
    return x2[:N]
